# Initial kernel scaffold; baseline (speedup 1.0000x reference)
import jax
import jax.numpy as jnp
from jax import lax
from jax.experimental import pallas as pl
from jax.experimental.pallas import tpu as pltpu

_K = 10
_BB = 4


def _body(x_ref, sample_ref, distr_ref, ent_ref, ent_acc):
    i = pl.program_id(0)
    x = x_ref[...]
    bb, n = x.shape
    c = jnp.abs(x)
    z0f = (x > 0).astype(jnp.float32)

    col = lax.broadcasted_iota(jnp.int32, (bb, n), 1)

    cw = c
    c_small = []
    idxs = []
    for _ in range(_K):
        m = jnp.min(cw, axis=1, keepdims=True)
        idx = jnp.min(jnp.where(cw == m, col, n), axis=1, keepdims=True)
        c_small.append(m)
        idxs.append(idx)
        cw = jnp.where(col == idx, jnp.inf, cw)

    csel = jnp.zeros((bb, n), jnp.int32)
    hit = jnp.zeros((bb, n), jnp.int32)
    for t in range(_K):
        eq = col == idxs[t]
        csel = jnp.where(eq, t, csel)
        hit = hit | eq.astype(jnp.int32)

    ns = 1 << _K
    sio = lax.broadcasted_iota(jnp.int32, (bb, ns), 1)
    sums = jnp.zeros((bb, ns), jnp.float32)
    for t in range(_K):
        bit = ((sio >> t) & 1).astype(jnp.float32)
        sums = sums + bit * c_small[t]

    sw = sums
    best = []
    bsum = []
    for _ in range(_K):
        m = jnp.min(sw, axis=1, keepdims=True)
        bidx = jnp.min(jnp.where(sw == m, sio, ns), axis=1, keepdims=True)
        best.append(bidx)
        bsum.append(m)
        sw = jnp.where(sio == bidx, jnp.inf, sw)

    for j in range(_K):
        fl = (jnp.right_shift(best[j], csel) & 1) * hit
        flf = fl.astype(jnp.float32)
        sample_ref[:, j, :] = z0f + flf - 2.0 * z0f * flf

    s_pos = jnp.sum(jnp.maximum(x, 0.0), axis=1, keepdims=True)
    scores = s_pos - jnp.concatenate(bsum, axis=1)

    kio = lax.broadcasted_iota(jnp.int32, (bb, _K), 1)
    work = scores
    zs = []
    for _ in range(_K):
        m = jnp.max(work, axis=1, keepdims=True)
        im = jnp.min(jnp.where(work == m, kio, _K), axis=1, keepdims=True)
        work = jnp.where(kio == im, -jnp.inf, work)
        zs.append(m)
    css = jnp.zeros((bb, 1), jnp.float32)
    kf = jnp.zeros((bb, 1), jnp.float32)
    for t in range(_K):
        css = css + zs[t]
        sup = (1.0 + (t + 1) * zs[t]) > css
        kf = kf + sup.astype(jnp.float32)
    css_k = jnp.zeros((bb, 1), jnp.float32)
    css_run = jnp.zeros((bb, 1), jnp.float32)
    for t in range(_K):
        css_run = css_run + zs[t]
        css_k = jnp.where(kf == (t + 1), css_run, css_k)
    tau = (css_k - 1.0) / kf
    p = jnp.maximum(scores - tau, 0.0)
    distr_ref[pl.ds(i * bb, bb), :] = p

    safe = jnp.where(p > 0, p, 1.0)
    part = -jnp.sum(jnp.where(p > 0, p * jnp.log(safe), 0.0))
    prev = jnp.where(i == 0, 0.0, ent_acc[0])
    tot = prev + part
    ent_acc[0] = tot
    ent_ref[0, 0] = tot


def kernel(logits):
    b, n = logits.shape
    nsteps = b // _BB
    sample, distr, ent = pl.pallas_call(
        _body,
        grid=(nsteps,),
        in_specs=[pl.BlockSpec((_BB, n), lambda i: (i, 0))],
        out_specs=[
            pl.BlockSpec((_BB, _K, n), lambda i: (i, 0, 0)),
            pl.BlockSpec((b, _K), lambda i: (0, 0)),
            pl.BlockSpec((1, 1), lambda i: (0, 0)),
        ],
        out_shape=[
            jax.ShapeDtypeStruct((b, _K, n), jnp.float32),
            jax.ShapeDtypeStruct((b, _K), jnp.float32),
            jax.ShapeDtypeStruct((1, 1), jnp.float32),
        ],
        scratch_shapes=[pltpu.SMEM((1,), jnp.float32)],
    )(logits)
    return (sample, distr, (ent / b).reshape(()))


# speedup vs baseline: 1.0382x; 1.0382x over previous
import jax
import jax.numpy as jnp
from jax import lax
from jax.experimental import pallas as pl
from jax.experimental.pallas import tpu as pltpu

_K = 10
_BB = 8


def _sample_body(x_ref, sample_ref):
    x = x_ref[...]
    bb, n = x.shape
    c = jnp.abs(x)
    z0f = (x > 0).astype(jnp.float32)

    col = lax.broadcasted_iota(jnp.int32, (bb, n), 1)

    cw = c
    c_small = []
    idxs = []
    for _ in range(_K):
        m = jnp.min(cw, axis=1, keepdims=True)
        idx = jnp.min(jnp.where(cw == m, col, n), axis=1, keepdims=True)
        c_small.append(m)
        idxs.append(idx)
        cw = jnp.where(col == idx, jnp.inf, cw)

    csel = jnp.zeros((bb, n), jnp.int32)
    hit = jnp.zeros((bb, n), jnp.int32)
    for t in range(_K):
        eq = col == idxs[t]
        csel = jnp.where(eq, t, csel)
        hit = hit | eq.astype(jnp.int32)

    ns = 1 << _K
    sio = lax.broadcasted_iota(jnp.int32, (bb, ns), 1)
    sums = jnp.zeros((bb, ns), jnp.float32)
    for t in range(_K):
        bit = ((sio >> t) & 1).astype(jnp.float32)
        sums = sums + bit * c_small[t]

    sw = sums
    best = []
    for _ in range(_K):
        m = jnp.min(sw, axis=1, keepdims=True)
        bidx = jnp.min(jnp.where(sw == m, sio, ns), axis=1, keepdims=True)
        best.append(bidx)
        sw = jnp.where(sio == bidx, jnp.inf, sw)

    for j in range(_K):
        fl = (jnp.right_shift(best[j], csel) & 1) * hit
        flf = fl.astype(jnp.float32)
        sample_ref[:, j, :] = z0f + flf - 2.0 * z0f * flf


def _distr_body(s_ref, distr_ref, ent_ref):
    scores = s_ref[...]
    b, k = scores.shape
    kio = lax.broadcasted_iota(jnp.int32, (b, k), 1)
    work = scores
    zs = []
    for _ in range(k):
        m = jnp.max(work, axis=1, keepdims=True)
        im = jnp.min(jnp.where(work == m, kio, k), axis=1, keepdims=True)
        work = jnp.where(kio == im, -jnp.inf, work)
        zs.append(m)
    css = jnp.zeros((b, 1), jnp.float32)
    kf = jnp.zeros((b, 1), jnp.float32)
    for t in range(k):
        css = css + zs[t]
        sup = (1.0 + (t + 1) * zs[t]) > css
        kf = kf + sup.astype(jnp.float32)
    css_k = jnp.zeros((b, 1), jnp.float32)
    css_run = jnp.zeros((b, 1), jnp.float32)
    for t in range(k):
        css_run = css_run + zs[t]
        css_k = jnp.where(kf == (t + 1), css_run, css_k)
    tau = (css_k - 1.0) / kf
    p = jnp.maximum(scores - tau, 0.0)
    distr_ref[...] = p

    safe = jnp.where(p > 0, p, 1.0)
    ent = -jnp.sum(jnp.where(p > 0, p * jnp.log(safe), 0.0))
    ent_ref[...] = jnp.broadcast_to(ent / b, (1, 1))


def kernel(logits):
    b, n = logits.shape
    nsteps = b // _BB
    sample = pl.pallas_call(
        _sample_body,
        grid=(nsteps,),
        in_specs=[pl.BlockSpec((_BB, n), lambda i: (i, 0))],
        out_specs=pl.BlockSpec((_BB, _K, n), lambda i: (i, 0, 0)),
        out_shape=jax.ShapeDtypeStruct((b, _K, n), jnp.float32),
    )(logits)
    scores = jnp.einsum('bkj,bj->bk', sample, logits)
    distr, ent = pl.pallas_call(
        _distr_body,
        out_shape=[
            jax.ShapeDtypeStruct((b, _K), jnp.float32),
            jax.ShapeDtypeStruct((1, 1), jnp.float32),
        ],
    )(scores)
    return (sample, distr, ent.reshape(()))


# speedup vs baseline: 1.1736x; 1.1304x over previous
import jax
import jax.numpy as jnp
from jax import lax
from jax.experimental import pallas as pl
from jax.experimental.pallas import tpu as pltpu

_K = 10
_BB = 8


def _sample_body(x_ref, sample_ref):
    x = x_ref[...]
    bb, n = x.shape
    c = jnp.abs(x)

    col = lax.broadcasted_iota(jnp.int32, (bb, n), 1)

    cw = c
    c_small = []
    v = jnp.zeros((bb, n), jnp.int32)
    for t in range(_K):
        m = jnp.min(cw, axis=1, keepdims=True)
        idx = jnp.min(jnp.where(cw == m, col, n), axis=1, keepdims=True)
        c_small.append(m)
        eq = col == idx
        cw = jnp.where(eq, jnp.inf, cw)
        v = jnp.where(eq, 1 << t, v)

    ns = 1 << _K
    sio = lax.broadcasted_iota(jnp.int32, (bb, ns), 1)
    sums = jnp.zeros((bb, ns), jnp.float32)
    for t in range(_K):
        bit = ((sio >> t) & 1).astype(jnp.float32)
        sums = sums + bit * c_small[t]

    sw = sums
    best = []
    for _ in range(_K):
        m = jnp.min(sw, axis=1, keepdims=True)
        bidx = jnp.min(jnp.where(sw == m, sio, ns), axis=1, keepdims=True)
        best.append(bidx)
        sw = jnp.where(sio == bidx, jnp.inf, sw)

    z0i = (x > 0).astype(jnp.int32)
    for j in range(_K):
        fl = (best[j] & v) > 0
        sample_ref[:, j, :] = (z0i ^ fl.astype(jnp.int32)).astype(jnp.float32)


def _distr_body(s_ref, distr_ref, ent_ref):
    scores = s_ref[...]
    b, k = scores.shape
    kio = lax.broadcasted_iota(jnp.int32, (b, k), 1)
    work = scores
    zs = []
    for _ in range(k):
        m = jnp.max(work, axis=1, keepdims=True)
        im = jnp.min(jnp.where(work == m, kio, k), axis=1, keepdims=True)
        work = jnp.where(kio == im, -jnp.inf, work)
        zs.append(m)
    css = jnp.zeros((b, 1), jnp.float32)
    kf = jnp.zeros((b, 1), jnp.float32)
    for t in range(k):
        css = css + zs[t]
        sup = (1.0 + (t + 1) * zs[t]) > css
        kf = kf + sup.astype(jnp.float32)
    css_k = jnp.zeros((b, 1), jnp.float32)
    css_run = jnp.zeros((b, 1), jnp.float32)
    for t in range(k):
        css_run = css_run + zs[t]
        css_k = jnp.where(kf == (t + 1), css_run, css_k)
    tau = (css_k - 1.0) / kf
    p = jnp.maximum(scores - tau, 0.0)
    distr_ref[...] = p

    safe = jnp.where(p > 0, p, 1.0)
    ent = -jnp.sum(jnp.where(p > 0, p * jnp.log(safe), 0.0))
    ent_ref[...] = jnp.broadcast_to(ent / b, (1, 1))


def kernel(logits):
    b, n = logits.shape
    nsteps = b // _BB
    sample = pl.pallas_call(
        _sample_body,
        grid=(nsteps,),
        in_specs=[pl.BlockSpec((_BB, n), lambda i: (i, 0))],
        out_specs=pl.BlockSpec((_BB, _K, n), lambda i: (i, 0, 0)),
        out_shape=jax.ShapeDtypeStruct((b, _K, n), jnp.float32),
    )(logits)
    scores = jnp.einsum('bkj,bj->bk', sample, logits)
    distr, ent = pl.pallas_call(
        _distr_body,
        out_shape=[
            jax.ShapeDtypeStruct((b, _K), jnp.float32),
            jax.ShapeDtypeStruct((1, 1), jnp.float32),
        ],
    )(scores)
    return (sample, distr, ent.reshape(()))


# speedup vs baseline: 1.1989x; 1.0216x over previous
import jax
import jax.numpy as jnp
from jax import lax
from jax.experimental import pallas as pl
from jax.experimental.pallas import tpu as pltpu

_K = 10
_BB = 8


def _sample_body(x_ref, sample_ref):
    x = x_ref[...]
    bb, n = x.shape
    c = jnp.abs(x)

    col = lax.broadcasted_iota(jnp.int32, (bb, n), 1)

    cw = c
    c_small = []
    v = jnp.zeros((bb, n), jnp.int32)
    for t in range(_K):
        m = jnp.min(cw, axis=1, keepdims=True)
        idx = jnp.min(jnp.where(cw == m, col, n), axis=1, keepdims=True)
        c_small.append(m)
        eq = col == idx
        cw = jnp.where(eq, jnp.inf, cw)
        v = jnp.where(eq, 1 << t, v)

    ns = 1 << _K
    sio = lax.broadcasted_iota(jnp.int32, (bb, ns), 1)
    sums = jnp.zeros((bb, ns), jnp.float32)
    for t in range(_K):
        bit = ((sio >> t) & 1).astype(jnp.float32)
        sums = sums + bit * c_small[t]

    sw = sums
    best = []
    for _ in range(_K):
        m = jnp.min(sw, axis=1, keepdims=True)
        bidx = jnp.min(jnp.where(sw == m, sio, ns), axis=1, keepdims=True)
        best.append(bidx)
        sw = jnp.where(sio == bidx, jnp.inf, sw)

    z0f = (x > 0).astype(jnp.float32)
    z1f = 1.0 - z0f
    for j in range(_K):
        sample_ref[:, j, :] = jnp.where((best[j] & v) != 0, z1f, z0f)


def _distr_body(s_ref, distr_ref, ent_ref):
    scores = s_ref[...]
    b, k = scores.shape
    kio = lax.broadcasted_iota(jnp.int32, (b, k), 1)
    work = scores
    zs = []
    for _ in range(k):
        m = jnp.max(work, axis=1, keepdims=True)
        im = jnp.min(jnp.where(work == m, kio, k), axis=1, keepdims=True)
        work = jnp.where(kio == im, -jnp.inf, work)
        zs.append(m)
    css = jnp.zeros((b, 1), jnp.float32)
    kf = jnp.zeros((b, 1), jnp.float32)
    for t in range(k):
        css = css + zs[t]
        sup = (1.0 + (t + 1) * zs[t]) > css
        kf = kf + sup.astype(jnp.float32)
    css_k = jnp.zeros((b, 1), jnp.float32)
    css_run = jnp.zeros((b, 1), jnp.float32)
    for t in range(k):
        css_run = css_run + zs[t]
        css_k = jnp.where(kf == (t + 1), css_run, css_k)
    tau = (css_k - 1.0) / kf
    p = jnp.maximum(scores - tau, 0.0)
    distr_ref[...] = p

    safe = jnp.where(p > 0, p, 1.0)
    ent = -jnp.sum(jnp.where(p > 0, p * jnp.log(safe), 0.0))
    ent_ref[...] = jnp.broadcast_to(ent / b, (1, 1))


def kernel(logits):
    b, n = logits.shape
    nsteps = b // _BB
    sample = pl.pallas_call(
        _sample_body,
        grid=(nsteps,),
        in_specs=[pl.BlockSpec((_BB, n), lambda i: (i, 0))],
        out_specs=pl.BlockSpec((_BB, _K, n), lambda i: (i, 0, 0)),
        out_shape=jax.ShapeDtypeStruct((b, _K, n), jnp.float32),
    )(logits)
    scores = jnp.einsum('bkj,bj->bk', sample, logits)
    distr, ent = pl.pallas_call(
        _distr_body,
        out_shape=[
            jax.ShapeDtypeStruct((b, _K), jnp.float32),
            jax.ShapeDtypeStruct((1, 1), jnp.float32),
        ],
    )(scores)
    return (sample, distr, ent.reshape(()))
